# baseline (speedup 1.0000x reference)
import functools

import jax
import jax.numpy as jnp
from jax import lax
from jax.experimental import pallas as pl
from jax.experimental.pallas import tpu as pltpu
from jax.experimental.pallas import tpu_sc as plsc

B = 2
N = 16384
NPOINT = 2048
NSAMPLE = 32
IN_CH = 32
FREQ = 10

_R = 128
_C = 128

_QB = 128
_NB = 128
_CW = N // _NB
_ROUNDS = 5
_RPAD = 8
_INF = 1e30
_BIG = 1 << 30


def _fps_body(xyz_ref, xyzs_ref, newxyz_ref, dist_ref):
    rows = jax.lax.broadcasted_iota(jnp.int32, (_R, _C), 0)
    cols = jax.lax.broadcasted_iota(jnp.int32, (_R, _C), 1)
    flat = rows * _C + cols
    dist_ref[...] = jnp.full((B, _R, _C), 1e10, jnp.float32)
    init = ([jnp.int32(0) for _ in range(B)],
            [jnp.full((_R, _C), 1e10, jnp.float32) for _ in range(B)])

    def body(i, state):
        fs, dists = state
        newf, newd = [], []
        for b in range(B):
            f = fs[b]
            cx = xyzs_ref[b, 0, f]
            cy = xyzs_ref[b, 1, f]
            cz = xyzs_ref[b, 2, f]
            newxyz_ref[b, 0, i] = cx
            newxyz_ref[b, 1, i] = cy
            newxyz_ref[b, 2, i] = cz
            dx = xyz_ref[b, 0] - cx
            dy = xyz_ref[b, 1] - cy
            dz = xyz_ref[b, 2] - cz
            d = (dx * dx + dy * dy) + dz * dz
            dist = jnp.minimum(dists[b], d)
            m = jnp.max(dist)
            newf.append(
                jnp.min(jnp.where(dist == m, flat, N)).astype(jnp.int32))
            newd.append(dist)
        return (newf, newd)

    jax.lax.fori_loop(0, NPOINT, body, init)


def _fps(xyz):
    xyzt = xyz.transpose(0, 2, 1)
    xyz_r = xyzt.reshape(B, 3, _R, _C)
    newxyz_t = pl.pallas_call(
        _fps_body,
        grid=(1,),
        in_specs=[
            pl.BlockSpec((B, 3, _R, _C), lambda i: (0, 0, 0, 0)),
            pl.BlockSpec((B, 3, N), lambda i: (0, 0, 0),
                         memory_space=pltpu.SMEM),
        ],
        out_specs=pl.BlockSpec((B, 3, NPOINT), lambda i: (0, 0, 0),
                               memory_space=pltpu.SMEM),
        out_shape=jax.ShapeDtypeStruct((B, 3, NPOINT), jnp.float32),
        scratch_shapes=[pltpu.VMEM((B, _R, _C), jnp.float32)],
    )(xyz_r, xyzt)
    return newxyz_t


def _knn_body(xyzc_ref, q_ref, lhs_ref, rhs_ref, out_ref,
              d3_ref, cval_ref, cgid_ref):
    xn = xyzc_ref[0, 0][:, :, None]
    yn = xyzc_ref[0, 1][:, :, None]
    zn = xyzc_ref[0, 2][:, :, None]
    sqn = (xn * xn + yn * yn) + zn * zn
    qx = q_ref[0, 0][None, None, :]
    qy = q_ref[0, 1][None, None, :]
    qz = q_ref[0, 2][None, None, :]
    sqq = (qx * qx + qy * qy) + qz * qz
    cross = jnp.dot(lhs_ref[0], rhs_ref[0],
                    preferred_element_type=jnp.float32)
    d3_ref[...] = (sqq + sqn) - 2.0 * cross.reshape(_NB, _CW, _QB)

    ij = jax.lax.broadcasted_iota(jnp.int32, (_NB, _CW, _QB), 1)
    gbase = jax.lax.broadcasted_iota(jnp.int32, (_NB, _QB), 0) * _CW

    cval_ref[...] = jnp.full((_NB, _RPAD, _QB), _INF, jnp.float32)
    cgid_ref[...] = jnp.full((_NB, _RPAD, _QB), _BIG, jnp.int32)

    for r in range(_ROUNDS):
        d3 = d3_ref[...]
        m = jnp.min(d3, axis=1)
        jb = jnp.min(jnp.where(d3 == m[:, None, :], ij, _CW),
                     axis=1).astype(jnp.int32)
        cval_ref[:, pl.ds(r, 1), :] = m[:, None, :]
        cgid_ref[:, pl.ds(r, 1), :] = (gbase + jb)[:, None, :]
        if r + 1 < _ROUNDS:
            d3_ref[...] = jnp.where(ij == jb[:, None, :], _INF, d3)

    for t in range(NSAMPLE):
        cval = cval_ref[...]
        cgid = cgid_ref[...]
        m = jnp.min(jnp.min(cval, axis=0), axis=0)
        sel = jnp.min(jnp.min(
            jnp.where(cval == m[None, None, :], cgid, _BIG),
            axis=0), axis=0)
        out_ref[0, pl.ds(t, 1), :] = sel[None, :]
        cval_ref[...] = jnp.where(cgid == sel[None, None, :], _INF, cval)


def _knn(xyzc, newxyz_t, lhs_bf, rhs_bf):
    idx_t = pl.pallas_call(
        _knn_body,
        grid=(B, NPOINT // _QB),
        in_specs=[
            pl.BlockSpec((1, 3, _NB, _CW), lambda b, s: (b, 0, 0, 0)),
            pl.BlockSpec((1, 3, _QB), lambda b, s: (b, 0, s)),
            pl.BlockSpec((1, N, 8), lambda b, s: (b, 0, 0)),
            pl.BlockSpec((1, 8, _QB), lambda b, s: (b, 0, s)),
        ],
        out_specs=pl.BlockSpec((1, NSAMPLE, _QB), lambda b, s: (b, 0, s)),
        out_shape=jax.ShapeDtypeStruct((B, NSAMPLE, NPOINT), jnp.int32),
        scratch_shapes=[
            pltpu.VMEM((_NB, _CW, _QB), jnp.float32),
            pltpu.VMEM((_NB, _RPAD, _QB), jnp.float32),
            pltpu.VMEM((_NB, _RPAD, _QB), jnp.int32),
        ],
    )(xyzc, newxyz_t, lhs_bf, rhs_bf)
    return idx_t


_ROWS = B * NPOINT * NSAMPLE
_TW = 128


def _gather(table, gidx):
    info = plsc.get_sparse_core_info()
    nw = info.num_cores * info.num_subcores
    b_per_w = _ROWS // nw
    nch = 8
    c_rows = b_per_w // nch
    mesh = plsc.VectorSubcoreMesh(core_axis_name="c", subcore_axis_name="s")

    @functools.partial(
        pl.kernel, mesh=mesh,
        out_type=jax.ShapeDtypeStruct((_ROWS, _TW), jnp.float32),
        scratch_types=[
            pltpu.VMEM((c_rows,), jnp.int32),
            pltpu.VMEM((c_rows, _TW), jnp.float32),
            pltpu.SemaphoreType.DMA,
        ],
    )
    def k(table_hbm, idx_hbm, out_hbm, idx_v, rows_v, sem):
        wid = lax.axis_index("s") * info.num_cores + lax.axis_index("c")
        for ch in range(nch):
            base = wid * b_per_w + ch * c_rows
            pltpu.sync_copy(idx_hbm.at[pl.ds(base, c_rows)], idx_v)
            pltpu.async_copy(table_hbm.at[idx_v], rows_v, sem).wait()
            pltpu.sync_copy(rows_v, out_hbm.at[pl.ds(base, c_rows)])

    return k(table, gidx)


_QM = 128
_RW = _QM * NSAMPLE


def _ln(x, g, b, eps=1e-5):
    mu = x.mean(-1, keepdims=True)
    var = ((x - mu) ** 2).mean(-1, keepdims=True)
    return (x - mu) / jnp.sqrt(var + eps) * g + b


def _mlp_body(rows_ref, nq_ref, w1a_ref, w1b_ref, b1_ref, g1_ref, be1_ref,
              w2_ref, b2_ref, g2_ref, be2_ref, w3_ref, b3_ref, out_ref):
    fi = jax.lax.broadcasted_iota(jnp.int32, (1, FREQ), 1)
    freqs = (1 << fi).astype(jnp.float32)
    parts = []
    for c in range(3):
        rel_c = rows_ref[:, IN_CH + c:IN_CH + c + 1] - nq_ref[:, c:c + 1]
        th = rel_c * freqs
        parts.append(jnp.sin(th))
        parts.append(jnp.cos(th))
    enc = jnp.concatenate(parts, axis=-1)
    f_bf = rows_ref[:, :IN_CH].astype(jnp.bfloat16)
    h = (jnp.dot(f_bf, w1a_ref[...], preferred_element_type=jnp.float32)
         + jnp.dot(enc.astype(jnp.bfloat16), w1b_ref[...],
                   preferred_element_type=jnp.float32))
    h = h + b1_ref[...]
    h = jax.nn.relu(_ln(h, g1_ref[...], be1_ref[...]))
    h = jnp.dot(h.astype(jnp.bfloat16), w2_ref[...],
                preferred_element_type=jnp.float32) + b2_ref[...]
    h = jax.nn.relu(_ln(h, g2_ref[...], be2_ref[...]))
    h = jnp.dot(h.astype(jnp.bfloat16), w3_ref[...],
                preferred_element_type=jnp.float32) + b3_ref[...]
    out_ref[...] = jnp.max(h.reshape(_QM, NSAMPLE, DIMS3), axis=1)


DIMS3 = 128


def _mlp(rows, nq_rep, W1, b1, g1, be1, W2, b2, g2, be2, W3, b3):
    grid = (_ROWS // _RW,)
    cw = lambda i: (0, 0)
    out = pl.pallas_call(
        _mlp_body,
        grid=grid,
        in_specs=[
            pl.BlockSpec((_RW, _TW), lambda i: (i, 0)),
            pl.BlockSpec((_RW, 3), lambda i: (i, 0)),
            pl.BlockSpec((IN_CH, 64), cw),
            pl.BlockSpec((60, 64), cw),
            pl.BlockSpec((1, 64), cw),
            pl.BlockSpec((1, 64), cw),
            pl.BlockSpec((1, 64), cw),
            pl.BlockSpec((64, 64), cw),
            pl.BlockSpec((1, 64), cw),
            pl.BlockSpec((1, 64), cw),
            pl.BlockSpec((1, 64), cw),
            pl.BlockSpec((64, DIMS3), cw),
            pl.BlockSpec((1, DIMS3), cw),
        ],
        out_specs=pl.BlockSpec((_QM, DIMS3), lambda i: (i, 0)),
        out_shape=jax.ShapeDtypeStruct((B * NPOINT, DIMS3), jnp.float32),
    )(rows, nq_rep,
      W1[:IN_CH].astype(jnp.bfloat16), W1[IN_CH:].astype(jnp.bfloat16),
      b1[None], g1[None], be1[None],
      W2.astype(jnp.bfloat16), b2[None], g2[None], be2[None],
      W3.astype(jnp.bfloat16), b3[None])
    return out


def kernel(xyz, features, W1, b1, g1, be1, W2, b2, g2, be2, W3, b3):
    newxyz_t = _fps(xyz)
    new_xyz = newxyz_t.transpose(0, 2, 1)

    xyzc = xyz.transpose(0, 2, 1).reshape(B, 3, _NB, _CW)
    lhs_bf = jnp.concatenate(
        [xyz, jnp.zeros((B, N, 5), jnp.float32)], axis=-1
    ).astype(jnp.bfloat16)
    rhs_bf = jnp.concatenate(
        [newxyz_t, jnp.zeros((B, 5, NPOINT), jnp.float32)], axis=1
    ).astype(jnp.bfloat16)
    knn_idx = _knn(xyzc, newxyz_t, lhs_bf, rhs_bf).transpose(0, 2, 1)

    table = jnp.concatenate(
        [features, xyz, jnp.zeros((B, N, _TW - IN_CH - 3), jnp.float32)],
        axis=-1).reshape(B * N, _TW)
    gidx = (knn_idx + jnp.arange(B, dtype=jnp.int32)[:, None, None] * N
            ).reshape(-1)
    rows = _gather(table, gidx)
    nq_rep = jnp.repeat(new_xyz.reshape(-1, 3), NSAMPLE, axis=0)
    new_features = _mlp(rows, nq_rep, W1, b1, g1, be1,
                        W2, b2, g2, be2, W3, b3).reshape(B, NPOINT, DIMS3)
    return (new_xyz, new_features)


# speedup vs baseline: 1.0034x; 1.0034x over previous
import functools

import jax
import jax.numpy as jnp
from jax import lax
from jax.experimental import pallas as pl
from jax.experimental.pallas import tpu as pltpu
from jax.experimental.pallas import tpu_sc as plsc

B = 2
N = 16384
NPOINT = 2048
NSAMPLE = 32
IN_CH = 32
FREQ = 10

_R = 128
_C = 128

_QB = 128
_NB = 128
_CW = N // _NB
_ROUNDS = 5
_RPAD = 8
_INF = 1e30
_BIG = 1 << 30


def _fps_body(xyz_ref, xyzs_ref, newxyz_ref, dist_ref):
    rows = jax.lax.broadcasted_iota(jnp.int32, (_R, _C), 0)
    cols = jax.lax.broadcasted_iota(jnp.int32, (_R, _C), 1)
    flat = rows * _C + cols
    dist_ref[...] = jnp.full((B, _R, _C), 1e10, jnp.float32)

    def body(i, fs):
        newf = []
        for b in range(B):
            f = fs[b]
            cx = xyzs_ref[b, 0, f]
            cy = xyzs_ref[b, 1, f]
            cz = xyzs_ref[b, 2, f]
            newxyz_ref[b, 0, i] = cx
            newxyz_ref[b, 1, i] = cy
            newxyz_ref[b, 2, i] = cz
            dx = xyz_ref[b, 0] - cx
            dy = xyz_ref[b, 1] - cy
            dz = xyz_ref[b, 2] - cz
            d = (dx * dx + dy * dy) + dz * dz
            dist = jnp.minimum(dist_ref[b], d)
            dist_ref[b] = dist
            m = jnp.max(dist)
            newf.append(
                jnp.min(jnp.where(dist == m, flat, N)).astype(jnp.int32))
        return tuple(newf)

    jax.lax.fori_loop(0, NPOINT, body, tuple(jnp.int32(0) for _ in range(B)))


def _fps(xyz):
    xyzt = xyz.transpose(0, 2, 1)
    xyz_r = xyzt.reshape(B, 3, _R, _C)
    newxyz_t = pl.pallas_call(
        _fps_body,
        grid=(1,),
        in_specs=[
            pl.BlockSpec((B, 3, _R, _C), lambda i: (0, 0, 0, 0)),
            pl.BlockSpec((B, 3, N), lambda i: (0, 0, 0),
                         memory_space=pltpu.SMEM),
        ],
        out_specs=pl.BlockSpec((B, 3, NPOINT), lambda i: (0, 0, 0),
                               memory_space=pltpu.SMEM),
        out_shape=jax.ShapeDtypeStruct((B, 3, NPOINT), jnp.float32),
        scratch_shapes=[pltpu.VMEM((B, _R, _C), jnp.float32)],
    )(xyz_r, xyzt)
    return newxyz_t


def _knn_body(xyzc_ref, q_ref, lhs_ref, rhs_ref, out_ref,
              d3_ref, cval_ref, cgid_ref):
    xn = xyzc_ref[0, 0][:, :, None]
    yn = xyzc_ref[0, 1][:, :, None]
    zn = xyzc_ref[0, 2][:, :, None]
    sqn = (xn * xn + yn * yn) + zn * zn
    qx = q_ref[0, 0][None, None, :]
    qy = q_ref[0, 1][None, None, :]
    qz = q_ref[0, 2][None, None, :]
    sqq = (qx * qx + qy * qy) + qz * qz
    cross = jnp.dot(lhs_ref[0], rhs_ref[0],
                    preferred_element_type=jnp.float32)
    d3_ref[...] = (sqq + sqn) - 2.0 * cross.reshape(_NB, _CW, _QB)

    ij = jax.lax.broadcasted_iota(jnp.int32, (_NB, _CW, _QB), 1)
    gbase = jax.lax.broadcasted_iota(jnp.int32, (_NB, _QB), 0) * _CW

    cval_ref[...] = jnp.full((_NB, _RPAD, _QB), _INF, jnp.float32)
    cgid_ref[...] = jnp.full((_NB, _RPAD, _QB), _BIG, jnp.int32)

    for r in range(_ROUNDS):
        d3 = d3_ref[...]
        m = jnp.min(d3, axis=1)
        jb = jnp.min(jnp.where(d3 == m[:, None, :], ij, _CW),
                     axis=1).astype(jnp.int32)
        cval_ref[:, pl.ds(r, 1), :] = m[:, None, :]
        cgid_ref[:, pl.ds(r, 1), :] = (gbase + jb)[:, None, :]
        if r + 1 < _ROUNDS:
            d3_ref[...] = jnp.where(ij == jb[:, None, :], _INF, d3)

    for t in range(NSAMPLE):
        cval = cval_ref[...]
        cgid = cgid_ref[...]
        m = jnp.min(jnp.min(cval, axis=0), axis=0)
        sel = jnp.min(jnp.min(
            jnp.where(cval == m[None, None, :], cgid, _BIG),
            axis=0), axis=0)
        out_ref[0, pl.ds(t, 1), :] = sel[None, :]
        cval_ref[...] = jnp.where(cgid == sel[None, None, :], _INF, cval)


def _knn(xyzc, newxyz_t, lhs_bf, rhs_bf):
    idx_t = pl.pallas_call(
        _knn_body,
        grid=(B, NPOINT // _QB),
        in_specs=[
            pl.BlockSpec((1, 3, _NB, _CW), lambda b, s: (b, 0, 0, 0)),
            pl.BlockSpec((1, 3, _QB), lambda b, s: (b, 0, s)),
            pl.BlockSpec((1, N, 8), lambda b, s: (b, 0, 0)),
            pl.BlockSpec((1, 8, _QB), lambda b, s: (b, 0, s)),
        ],
        out_specs=pl.BlockSpec((1, NSAMPLE, _QB), lambda b, s: (b, 0, s)),
        out_shape=jax.ShapeDtypeStruct((B, NSAMPLE, NPOINT), jnp.int32),
        scratch_shapes=[
            pltpu.VMEM((_NB, _CW, _QB), jnp.float32),
            pltpu.VMEM((_NB, _RPAD, _QB), jnp.float32),
            pltpu.VMEM((_NB, _RPAD, _QB), jnp.int32),
        ],
    )(xyzc, newxyz_t, lhs_bf, rhs_bf)
    return idx_t


_ROWS = B * NPOINT * NSAMPLE
_TW = 128


def _gather(table, gidx):
    info = plsc.get_sparse_core_info()
    nw = info.num_cores * info.num_subcores
    b_per_w = _ROWS // nw
    nch = 8
    c_rows = b_per_w // nch
    mesh = plsc.VectorSubcoreMesh(core_axis_name="c", subcore_axis_name="s")

    @functools.partial(
        pl.kernel, mesh=mesh,
        out_type=jax.ShapeDtypeStruct((_ROWS, _TW), jnp.float32),
        scratch_types=[
            pltpu.VMEM((c_rows,), jnp.int32),
            pltpu.VMEM((c_rows, _TW), jnp.float32),
            pltpu.SemaphoreType.DMA,
        ],
    )
    def k(table_hbm, idx_hbm, out_hbm, idx_v, rows_v, sem):
        wid = lax.axis_index("s") * info.num_cores + lax.axis_index("c")
        for ch in range(nch):
            base = wid * b_per_w + ch * c_rows
            pltpu.sync_copy(idx_hbm.at[pl.ds(base, c_rows)], idx_v)
            pltpu.async_copy(table_hbm.at[idx_v], rows_v, sem).wait()
            pltpu.sync_copy(rows_v, out_hbm.at[pl.ds(base, c_rows)])

    return k(table, gidx)


_QM = 128
_RW = _QM * NSAMPLE


def _ln(x, g, b, eps=1e-5):
    mu = x.mean(-1, keepdims=True)
    var = ((x - mu) ** 2).mean(-1, keepdims=True)
    return (x - mu) / jnp.sqrt(var + eps) * g + b


def _mlp_body(rows_ref, nq_ref, w1a_ref, w1b_ref, b1_ref, g1_ref, be1_ref,
              w2_ref, b2_ref, g2_ref, be2_ref, w3_ref, b3_ref, out_ref):
    fi = jax.lax.broadcasted_iota(jnp.int32, (1, FREQ), 1)
    freqs = (1 << fi).astype(jnp.float32)
    parts = []
    for c in range(3):
        rel_c = rows_ref[:, IN_CH + c:IN_CH + c + 1] - nq_ref[:, c:c + 1]
        th = rel_c * freqs
        parts.append(jnp.sin(th))
        parts.append(jnp.cos(th))
    enc = jnp.concatenate(parts, axis=-1)
    f_bf = rows_ref[:, :IN_CH].astype(jnp.bfloat16)
    h = (jnp.dot(f_bf, w1a_ref[...], preferred_element_type=jnp.float32)
         + jnp.dot(enc.astype(jnp.bfloat16), w1b_ref[...],
                   preferred_element_type=jnp.float32))
    h = h + b1_ref[...]
    h = jax.nn.relu(_ln(h, g1_ref[...], be1_ref[...]))
    h = jnp.dot(h.astype(jnp.bfloat16), w2_ref[...],
                preferred_element_type=jnp.float32) + b2_ref[...]
    h = jax.nn.relu(_ln(h, g2_ref[...], be2_ref[...]))
    h = jnp.dot(h.astype(jnp.bfloat16), w3_ref[...],
                preferred_element_type=jnp.float32) + b3_ref[...]
    out_ref[...] = jnp.max(h.reshape(_QM, NSAMPLE, DIMS3), axis=1)


DIMS3 = 128


def _mlp(rows, nq_rep, W1, b1, g1, be1, W2, b2, g2, be2, W3, b3):
    grid = (_ROWS // _RW,)
    cw = lambda i: (0, 0)
    out = pl.pallas_call(
        _mlp_body,
        grid=grid,
        in_specs=[
            pl.BlockSpec((_RW, _TW), lambda i: (i, 0)),
            pl.BlockSpec((_RW, 3), lambda i: (i, 0)),
            pl.BlockSpec((IN_CH, 64), cw),
            pl.BlockSpec((60, 64), cw),
            pl.BlockSpec((1, 64), cw),
            pl.BlockSpec((1, 64), cw),
            pl.BlockSpec((1, 64), cw),
            pl.BlockSpec((64, 64), cw),
            pl.BlockSpec((1, 64), cw),
            pl.BlockSpec((1, 64), cw),
            pl.BlockSpec((1, 64), cw),
            pl.BlockSpec((64, DIMS3), cw),
            pl.BlockSpec((1, DIMS3), cw),
        ],
        out_specs=pl.BlockSpec((_QM, DIMS3), lambda i: (i, 0)),
        out_shape=jax.ShapeDtypeStruct((B * NPOINT, DIMS3), jnp.float32),
    )(rows, nq_rep,
      W1[:IN_CH].astype(jnp.bfloat16), W1[IN_CH:].astype(jnp.bfloat16),
      b1[None], g1[None], be1[None],
      W2.astype(jnp.bfloat16), b2[None], g2[None], be2[None],
      W3.astype(jnp.bfloat16), b3[None])
    return out


def kernel(xyz, features, W1, b1, g1, be1, W2, b2, g2, be2, W3, b3):
    newxyz_t = _fps(xyz)
    new_xyz = newxyz_t.transpose(0, 2, 1)

    xyzc = xyz.transpose(0, 2, 1).reshape(B, 3, _NB, _CW)
    lhs_bf = jnp.concatenate(
        [xyz, jnp.zeros((B, N, 5), jnp.float32)], axis=-1
    ).astype(jnp.bfloat16)
    rhs_bf = jnp.concatenate(
        [newxyz_t, jnp.zeros((B, 5, NPOINT), jnp.float32)], axis=1
    ).astype(jnp.bfloat16)
    knn_idx = _knn(xyzc, newxyz_t, lhs_bf, rhs_bf).transpose(0, 2, 1)

    table = jnp.concatenate(
        [features, xyz, jnp.zeros((B, N, _TW - IN_CH - 3), jnp.float32)],
        axis=-1).reshape(B * N, _TW)
    gidx = (knn_idx + jnp.arange(B, dtype=jnp.int32)[:, None, None] * N
            ).reshape(-1)
    rows = _gather(table, gidx)
    nq_rep = jnp.repeat(new_xyz.reshape(-1, 3), NSAMPLE, axis=0)
    new_features = _mlp(rows, nq_rep, W1, b1, g1, be1,
                        W2, b2, g2, be2, W3, b3).reshape(B, NPOINT, DIMS3)
    return (new_xyz, new_features)
